# Initial kernel scaffold; baseline (speedup 1.0000x reference)
import jax
import jax.numpy as jnp
from jax.experimental import pallas as pl


def kernel(x, edge_index, W, b):
    raise NotImplementedError("write your pallas kernel here")


# speedup vs baseline: 13.2456x; 13.2456x over previous
import functools

import jax
import jax.numpy as jnp
from jax import lax
from jax.experimental import pallas as pl
from jax.experimental.pallas import tpu as pltpu
from jax.experimental.pallas import tpu_sc as plsc

D = 128
NC = 2
NS = 16
NW = NC * NS
CHUNK = 128


def _matmul_body(x_ref, w_ref, o_ref):
    o_ref[...] = jnp.dot(x_ref[...], w_ref[...],
                         preferred_element_type=jnp.float32)


def _finalize_body(acc_ref, cnt_ref, b_ref, o_ref):
    a = acc_ref[0, :, :] + acc_ref[1, :, :]
    cnt = jnp.sum(cnt_ref[...], axis=1, keepdims=True)
    scale = lax.rsqrt(cnt + 1.0)
    o_ref[...] = jnp.maximum(a * scale + b_ref[...], 0.0)


def _make_sc_scatter(n_chunks, n_acc, n_deg):
    mesh = plsc.VectorSubcoreMesh(core_axis_name="c", subcore_axis_name="s")
    per_tile = n_acc // NS

    @functools.partial(
        pl.kernel,
        mesh=mesh,
        compiler_params=pltpu.CompilerParams(needs_layout_passes=False),
        out_type=[
            jax.ShapeDtypeStruct((NC, n_acc, D), jnp.float32),
            jax.ShapeDtypeStruct((NW, n_deg), jnp.float32),
        ],
        scratch_types=[
            pltpu.VMEM((4, CHUNK), jnp.int32),
            pltpu.VMEM((4, CHUNK), jnp.int32),
            pltpu.VMEM((CHUNK, D), jnp.float32),
            pltpu.VMEM((CHUNK, D), jnp.float32),
            pltpu.VMEM((n_deg,), jnp.float32),
            pltpu.VMEM_SHARED((n_acc, D), jnp.float32),
            pltpu.SemaphoreType.DMA,
            pltpu.SemaphoreType.DMA,
            pltpu.SemaphoreType.DMA,
        ],
    )
    def sc_scatter(src_hbm, dst_hbm, h_hbm, acc_hbm, cnt_hbm,
                   sidx, didx, rows0, rows1, deg_v, acc_sh,
                   gsem, isems, isemd):
        cid = lax.axis_index("c")
        sid = lax.axis_index("s")
        wid = cid * NS + sid

        def _zero_row(i, carry):
            for v in range(D // 16):
                rows0[i, pl.ds(v * 16, 16)] = jnp.zeros((16,), jnp.float32)
            return carry
        lax.fori_loop(0, CHUNK, _zero_row, 0)

        def _zero_deg(i, carry):
            deg_v[pl.ds(i * 16, 16)] = jnp.zeros((16,), jnp.float32)
            return carry
        lax.fori_loop(0, n_deg // 16, _zero_deg, 0)

        zbase = sid * per_tile
        for k in range(per_tile // CHUNK):
            pltpu.sync_copy(rows0, acc_sh.at[pl.ds(zbase + k * CHUNK, CHUNK)])
        plsc.subcore_barrier()

        for c in range(4):
            pltpu.async_copy(src_hbm.at[wid, c], sidx.at[c], isems)
            pltpu.async_copy(dst_hbm.at[wid, c], didx.at[c], isemd)
        for c in range(2):
            pltpu.make_async_copy(src_hbm.at[wid, 0], sidx.at[c], isems).wait()
            pltpu.make_async_copy(dst_hbm.at[wid, 0], didx.at[c], isemd).wait()
        pltpu.async_copy(h_hbm.at[sidx.at[0]], rows0, gsem)
        pltpu.async_copy(h_hbm.at[sidx.at[1]], rows1, gsem)

        rows = (rows0, rows1)
        ones16 = jnp.ones((16,), jnp.float32)
        rows_dummy = h_hbm.at[pl.ds(0, CHUNK)]

        def _quad(i, carry):
            for u in range(4):
                j = i * 4 + u
                buf = rows[u % 2]
                pltpu.make_async_copy(rows_dummy, buf, gsem).wait()
                pltpu.sync_copy(buf, acc_sh.at[didx.at[u]], add=True)
                for v in range(CHUNK // 16):
                    dv = didx[u, pl.ds(v * 16, 16)]
                    plsc.addupdate_scatter(deg_v, [dv], ones16)

                @pl.when(j + 4 < n_chunks)
                def _():
                    pltpu.async_copy(src_hbm.at[wid, j + 4], sidx.at[u], isems)
                    pltpu.async_copy(dst_hbm.at[wid, j + 4], didx.at[u], isemd)

                @pl.when(j + 2 < n_chunks)
                def _():
                    pltpu.make_async_copy(
                        src_hbm.at[wid, 0], sidx.at[u], isems).wait()
                    pltpu.make_async_copy(
                        dst_hbm.at[wid, 0], didx.at[u], isemd).wait()
                    pltpu.async_copy(
                        h_hbm.at[sidx.at[(u + 2) % 4]], buf, gsem)
            return carry
        lax.fori_loop(0, n_chunks // 4, _quad, 0)

        pltpu.sync_copy(deg_v, cnt_hbm.at[wid])
        plsc.subcore_barrier()

        pltpu.sync_copy(acc_sh.at[pl.ds(zbase, per_tile)],
                        acc_hbm.at[cid, pl.ds(zbase, per_tile)])

    return sc_scatter


def kernel(x, edge_index, W, b):
    n_src, d_in = x.shape
    n_out = n_src
    e = edge_index.shape[1]

    per_w = -(-e // NW)
    n_chunks = -(-(-(-per_w // CHUNK)) // 4) * 4
    per_w = n_chunks * CHUNK
    e_pad = per_w * NW

    trash = n_out
    n_acc = -(-(n_out + 1) // (NS * CHUNK)) * (NS * CHUNK)
    src = edge_index[0]
    dst = edge_index[1]
    pad = e_pad - e
    src_p = jnp.concatenate([src, jnp.zeros((pad,), jnp.int32)])
    dst_p = jnp.concatenate([dst, jnp.full((pad,), trash, jnp.int32)])
    src3 = src_p.reshape(NW, n_chunks, CHUNK)
    dst3 = dst_p.reshape(NW, n_chunks, CHUNK)

    blk = 2000
    h = pl.pallas_call(
        _matmul_body,
        grid=(n_src // blk,),
        in_specs=[
            pl.BlockSpec((blk, d_in), lambda i: (i, 0)),
            pl.BlockSpec((d_in, D), lambda i: (0, 0)),
        ],
        out_specs=pl.BlockSpec((blk, D), lambda i: (i, 0)),
        out_shape=jax.ShapeDtypeStruct((n_src, D), jnp.float32),
    )(x, W)

    n_deg = -(-(n_out + 1) // 16) * 16
    acc, cnt = _make_sc_scatter(n_chunks, n_acc, n_deg)(src3, dst3, h)
    cnt_t = cnt.T

    fblk = 2000
    out = pl.pallas_call(
        _finalize_body,
        grid=(n_out // fblk,),
        in_specs=[
            pl.BlockSpec((NC, fblk, D), lambda i: (0, i, 0)),
            pl.BlockSpec((fblk, NW), lambda i: (i, 0)),
            pl.BlockSpec((1, D), lambda i: (0, 0)),
        ],
        out_specs=pl.BlockSpec((fblk, D), lambda i: (i, 0)),
        out_shape=jax.ShapeDtypeStruct((n_out, D), jnp.float32),
    )(acc, cnt_t, b.reshape(1, D))
    return out
